# baseline (speedup 1.0000x reference)
import functools

import jax
import jax.numpy as jnp
from jax import lax
from jax.experimental import pallas as pl
from jax.experimental.pallas import tpu as pltpu
from jax.experimental.pallas import tpu_sc as plsc

N = 10000
E = 320000
D = 128
NPAD = 10240
NC, NS = 2, 16
NW = NC * NS
EPW = E // NW
K = 80
NCHUNK = EPW // K
NBUF = 2
RPT = NPAD // NS
BM = 1024
GRID = NPAD // BM

_f32 = jnp.float32
_mesh = plsc.VectorSubcoreMesh(core_axis_name="c", subcore_axis_name="s")


def _zero_fill(ref, n):
    zero16 = jnp.zeros((16,), _f32)

    def body(i, carry):
        ref[pl.ds(i * 16, 16)] = zero16
        return carry

    lax.fori_loop(0, n // 16, body, 0)


@functools.partial(
    pl.kernel,
    out_type=jax.ShapeDtypeStruct((NC * NPAD,), _f32),
    mesh=_mesh,
    scratch_types=[
        pltpu.VMEM_SHARED((NPAD,), _f32),
        pltpu.VMEM((EPW,), jnp.int32),
        pltpu.VMEM((K,), _f32),
        pltpu.VMEM((RPT,), _f32),
        pltpu.SemaphoreType.DMA,
    ],
)
def _deg_kernel(dst_hbm, out_hbm, deg_sh, idx_v, ones_v, zbuf_v, sem):
    cid = lax.axis_index("c")
    sid = lax.axis_index("s")
    wid = sid * NC + cid
    one16 = jnp.ones((16,), _f32)
    for i in range(K // 16):
        ones_v[pl.ds(i * 16, 16)] = one16
    _zero_fill(zbuf_v, RPT)
    pltpu.sync_copy(dst_hbm.at[pl.ds(wid * EPW, EPW)], idx_v)
    pltpu.sync_copy(zbuf_v, deg_sh.at[pl.ds(sid * RPT, RPT)])
    plsc.subcore_barrier()

    def fire(c, carry):
        pltpu.async_copy(ones_v, deg_sh.at[idx_v.at[pl.ds(c * K, K)]], sem,
                         add=True)
        return carry

    def drain(c, carry):
        pltpu.make_async_copy(out_hbm.at[pl.ds(0, K)], ones_v, sem).wait()
        return carry

    def batch(o, carry):
        lax.fori_loop(o * 25, (o + 1) * 25, fire, 0)
        lax.fori_loop(0, 25, drain, 0)
        return carry

    lax.fori_loop(0, NCHUNK // 25, batch, 0)
    plsc.subcore_barrier()
    pltpu.sync_copy(deg_sh.at[pl.ds(sid * RPT, RPT)],
                    out_hbm.at[pl.ds(cid * NPAD + sid * RPT, RPT)])


@functools.partial(
    pl.kernel,
    out_type=(jax.ShapeDtypeStruct((NC * NPAD, D), _f32),
              jax.ShapeDtypeStruct((NC * NPAD,), _f32)),
    mesh=_mesh,
    scratch_types=[
        pltpu.VMEM_SHARED((NPAD, D), _f32),
        pltpu.VMEM_SHARED((NPAD,), _f32),
        pltpu.VMEM((EPW,), jnp.int32),
        pltpu.VMEM((EPW,), jnp.int32),
        [pltpu.VMEM((K, D), _f32) for _ in range(NBUF)],
        [pltpu.VMEM((K,), _f32) for _ in range(NBUF)],
        pltpu.VMEM((RPT,), _f32),
        [pltpu.SemaphoreType.DMA for _ in range(NBUF)],
        [pltpu.SemaphoreType.DMA for _ in range(NBUF)],
        [pltpu.SemaphoreType.DMA for _ in range(NBUF)],
        [pltpu.SemaphoreType.DMA for _ in range(NBUF)],
    ],
)
def _scatter_kernel(src_hbm, dst_hbm, y_hbm, dinv_hbm, zeros_hbm,
                    acc_out, t_out,
                    acc_sh, t_sh, src_v, dst_v, rows, dval,
                    zbuf_v, semR, semD, semS, semT):
    cid = lax.axis_index("c")
    sid = lax.axis_index("s")
    wid = sid * NC + cid
    base = sid * RPT
    pltpu.sync_copy(zeros_hbm.at[pl.ds(base, RPT)], acc_sh.at[pl.ds(base, RPT)])
    _zero_fill(zbuf_v, RPT)
    pltpu.sync_copy(zbuf_v, t_sh.at[pl.ds(base, RPT)])
    pltpu.sync_copy(src_hbm.at[pl.ds(wid * EPW, EPW)], src_v)
    pltpu.sync_copy(dst_hbm.at[pl.ds(wid * EPW, EPW)], dst_v)
    plsc.subcore_barrier()

    def gather(c, b):
        pltpu.async_copy(y_hbm.at[src_v.at[pl.ds(c * K, K)]], rows[b], semR[b])
        pltpu.async_copy(dinv_hbm.at[dst_v.at[pl.ds(c * K, K)]], dval[b],
                         semD[b])

    def process(c, b):
        pltpu.make_async_copy(y_hbm.at[pl.ds(0, K)], rows[b], semR[b]).wait()
        pltpu.async_copy(rows[b], acc_sh.at[dst_v.at[pl.ds(c * K, K)]],
                         semS[b], add=True)
        pltpu.make_async_copy(dinv_hbm.at[pl.ds(0, K)], dval[b], semD[b]).wait()
        pltpu.async_copy(dval[b], t_sh.at[src_v.at[pl.ds(c * K, K)]],
                         semT[b], add=True)

    def drain_scatter(b):
        pltpu.make_async_copy(y_hbm.at[pl.ds(0, K)], rows[b], semS[b]).wait()
        pltpu.make_async_copy(dinv_hbm.at[pl.ds(0, K)], dval[b], semT[b]).wait()

    gather(0, 0)
    gather(1, 1)

    def body(g, carry):
        process(2 * g, 0)
        process(2 * g + 1, 1)
        drain_scatter(0)
        gather(2 * g + 2, 0)
        drain_scatter(1)
        gather(2 * g + 3, 1)
        return carry

    lax.fori_loop(0, (NCHUNK - 3) // 2, body, 0)
    process(NCHUNK - 3, 0)
    process(NCHUNK - 2, 1)
    drain_scatter(0)
    gather(NCHUNK - 1, 0)
    process(NCHUNK - 1, 0)
    drain_scatter(0)
    drain_scatter(1)
    plsc.subcore_barrier()
    pltpu.sync_copy(acc_sh.at[pl.ds(base, RPT)],
                    acc_out.at[pl.ds(cid * NPAD + base, RPT)])
    pltpu.sync_copy(t_sh.at[pl.ds(base, RPT)],
                    t_out.at[pl.ds(cid * NPAD + base, RPT)])


def _mm_body(x_ref, w_ref, o_ref):
    o_ref[...] = jnp.dot(x_ref[...], w_ref[...],
                         preferred_element_type=jnp.float32)


_mm = pl.pallas_call(
    _mm_body,
    grid=(GRID,),
    in_specs=[
        pl.BlockSpec((BM, D), lambda i: (i, 0)),
        pl.BlockSpec((D, D), lambda i: (0, 0)),
    ],
    out_specs=pl.BlockSpec((BM, D), lambda i: (i, 0)),
    out_shape=jax.ShapeDtypeStruct((NPAD, D), _f32),
)


def _scale_body(d0_ref, d1_ref, xw_ref, y_ref, dv_ref):
    deg = d0_ref[...] + d1_ref[...] + 1.0
    dinv = lax.rsqrt(deg)
    dv_ref[...] = dinv
    y_ref[...] = xw_ref[...] * dinv


_scale = pl.pallas_call(
    _scale_body,
    grid=(GRID,),
    in_specs=[
        pl.BlockSpec((BM, 1), lambda i: (i, 0)),
        pl.BlockSpec((BM, 1), lambda i: (i, 0)),
        pl.BlockSpec((BM, D), lambda i: (i, 0)),
    ],
    out_specs=[
        pl.BlockSpec((BM, D), lambda i: (i, 0)),
        pl.BlockSpec((BM, 1), lambda i: (i, 0)),
    ],
    out_shape=[
        jax.ShapeDtypeStruct((NPAD, D), _f32),
        jax.ShapeDtypeStruct((NPAD, 1), _f32),
    ],
)


def _final_body(acc_ref, y_ref, dv_ref, t0_ref, t1_ref, b1_ref,
                W2_ref, b2_ref, Wr1_ref, br1_ref, Wr2_ref, br2_ref,
                Wu_ref, bu_ref, pred_ref, unc_ref, v_ref):
    i = pl.program_id(0)
    acc = acc_ref[0] + acc_ref[1] + y_ref[...]
    dinv = dv_ref[...]
    h = jnp.maximum(acc * dinv + b1_ref[...], 0.0)
    t = t0_ref[...] + t1_ref[...] + dinv
    c = dinv * t
    rowid = lax.broadcasted_iota(jnp.int32, (BM, 1), 0) + i * BM
    c = jnp.where(rowid < N, c, 0.0)
    part = jnp.sum(c * h, axis=0, keepdims=True)

    @pl.when(i == 0)
    def _():
        v_ref[...] = jnp.zeros_like(v_ref)

    v_ref[...] += part

    @pl.when(i == GRID - 1)
    def _():
        v = v_ref[...] * (1.0 / N)
        g = jnp.dot(v, W2_ref[...], preferred_element_type=jnp.float32)
        g = g + b2_ref[...]
        hid = jnp.dot(g, Wr1_ref[...], preferred_element_type=jnp.float32)
        hid = jnp.maximum(hid + br1_ref[...], 0.0)
        pred = jnp.dot(hid, Wr2_ref[...], preferred_element_type=jnp.float32)
        pred_ref[...] = pred + br2_ref[...]
        u = jnp.dot(g, Wu_ref[...], preferred_element_type=jnp.float32)
        u = u + bu_ref[...]
        unc_ref[...] = 1.0 / (1.0 + jnp.exp(-u))


def _full(shape):
    return pl.BlockSpec(shape, lambda i: tuple(0 for _ in shape))


_final = pl.pallas_call(
    _final_body,
    grid=(GRID,),
    in_specs=[
        pl.BlockSpec((NC, BM, D), lambda i: (0, i, 0)),
        pl.BlockSpec((BM, D), lambda i: (i, 0)),
        pl.BlockSpec((BM, 1), lambda i: (i, 0)),
        pl.BlockSpec((BM, 1), lambda i: (i, 0)),
        pl.BlockSpec((BM, 1), lambda i: (i, 0)),
        _full((1, D)),
        _full((D, D)),
        _full((1, D)),
        _full((D, D // 2)),
        _full((1, D // 2)),
        _full((D // 2, 1)),
        _full((1, 1)),
        _full((D, 1)),
        _full((1, 1)),
    ],
    out_specs=[_full((1, 1)), _full((1, 1))],
    out_shape=[jax.ShapeDtypeStruct((1, 1), _f32),
               jax.ShapeDtypeStruct((1, 1), _f32)],
    scratch_shapes=[pltpu.VMEM((1, D), _f32)],
)


def kernel(x, edge_index, W1, b1, W2, b2, Wr1, br1, Wr2, br2, Wu, bu):
    src2 = edge_index[0]
    dst2 = edge_index[1]
    x_pad = jnp.pad(x, ((0, NPAD - N), (0, 0)))

    xw = _mm(x_pad, W1)
    deg2 = _deg_kernel(dst2)
    deg0 = deg2[:NPAD].reshape(NPAD, 1)
    deg1 = deg2[NPAD:].reshape(NPAD, 1)
    y, dinv_c = _scale(deg0, deg1, xw)
    dinv = dinv_c.reshape(NPAD)

    zeros2d = jnp.zeros((NPAD, D), _f32)
    acc2, t2 = _scatter_kernel(src2, dst2, y, dinv, zeros2d)
    acc3 = acc2.reshape(NC, NPAD, D)
    t0 = t2[:NPAD].reshape(NPAD, 1)
    t1 = t2[NPAD:].reshape(NPAD, 1)

    pred, unc = _final(acc3, y, dinv_c, t0, t1,
                       b1.reshape(1, D), W2, b2.reshape(1, D),
                       Wr1, br1.reshape(1, D // 2), Wr2, br2.reshape(1, 1),
                       Wu, bu.reshape(1, 1))
    return (pred, unc)


# speedup vs baseline: 1.0543x; 1.0543x over previous
import functools

import jax
import jax.numpy as jnp
from jax import lax
from jax.experimental import pallas as pl
from jax.experimental.pallas import tpu as pltpu
from jax.experimental.pallas import tpu_sc as plsc

N = 10000
E = 320000
D = 128
NPAD = 10240
NC, NS = 2, 16
NW = NC * NS
EPW = E // NW
K = 80
NCHUNK = EPW // K
RPT = NPAD // NS
BM = 1024
GRID = NPAD // BM

_f32 = jnp.float32
_mesh = plsc.VectorSubcoreMesh(core_axis_name="c", subcore_axis_name="s")


def _zero_fill(ref, n):
    zero16 = jnp.zeros((16,), _f32)

    def body(i, carry):
        ref[pl.ds(i * 16, 16)] = zero16
        return carry

    lax.fori_loop(0, n // 16, body, 0)


@functools.partial(
    pl.kernel,
    out_type=jax.ShapeDtypeStruct((NC * NPAD,), _f32),
    mesh=_mesh,
    scratch_types=[
        pltpu.VMEM_SHARED((NPAD,), _f32),
        pltpu.VMEM((EPW,), jnp.int32),
        pltpu.VMEM((K,), _f32),
        pltpu.VMEM((RPT,), _f32),
    ],
)
def _deg_kernel(dst_hbm, out_hbm, deg_sh, idx_v, ones_v, zbuf_v):
    cid = lax.axis_index("c")
    sid = lax.axis_index("s")
    wid = sid * NC + cid
    one16 = jnp.ones((16,), _f32)
    for i in range(K // 16):
        ones_v[pl.ds(i * 16, 16)] = one16
    _zero_fill(zbuf_v, RPT)
    pltpu.sync_copy(dst_hbm.at[pl.ds(wid * EPW, EPW)], idx_v)
    pltpu.sync_copy(zbuf_v, deg_sh.at[pl.ds(sid * RPT, RPT)])
    plsc.subcore_barrier()

    def chunk(c, carry):
        pltpu.sync_copy(ones_v, deg_sh.at[idx_v.at[pl.ds(c * K, K)]], add=True)
        return carry

    lax.fori_loop(0, NCHUNK, chunk, 0)
    plsc.subcore_barrier()
    pltpu.sync_copy(deg_sh.at[pl.ds(sid * RPT, RPT)],
                    out_hbm.at[pl.ds(cid * NPAD + sid * RPT, RPT)])


@functools.partial(
    pl.kernel,
    out_type=(jax.ShapeDtypeStruct((NC * NPAD, D), _f32),
              jax.ShapeDtypeStruct((NC * NPAD,), _f32)),
    mesh=_mesh,
    scratch_types=[
        pltpu.VMEM_SHARED((NPAD, D), _f32),
        pltpu.VMEM_SHARED((NPAD,), _f32),
        pltpu.VMEM((EPW,), jnp.int32),
        pltpu.VMEM((EPW,), jnp.int32),
        pltpu.VMEM((K, D), _f32),
        pltpu.VMEM((K, D), _f32),
        pltpu.VMEM((K,), _f32),
        pltpu.VMEM((K,), _f32),
        pltpu.VMEM((RPT,), _f32),
        pltpu.SemaphoreType.DMA,
        pltpu.SemaphoreType.DMA,
        pltpu.SemaphoreType.DMA,
        pltpu.SemaphoreType.DMA,
    ],
)
def _scatter_kernel(src_hbm, dst_hbm, y_hbm, dinv_hbm, zeros_hbm,
                    acc_out, t_out,
                    acc_sh, t_sh, src_v, dst_v, rowsA, rowsB,
                    dvalA, dvalB, zbuf_v, semA, semB, semA2, semB2):
    cid = lax.axis_index("c")
    sid = lax.axis_index("s")
    wid = sid * NC + cid
    base = sid * RPT
    pltpu.sync_copy(zeros_hbm.at[pl.ds(base, RPT)], acc_sh.at[pl.ds(base, RPT)])
    _zero_fill(zbuf_v, RPT)
    pltpu.sync_copy(zbuf_v, t_sh.at[pl.ds(base, RPT)])
    pltpu.sync_copy(src_hbm.at[pl.ds(wid * EPW, EPW)], src_v)
    pltpu.sync_copy(dst_hbm.at[pl.ds(wid * EPW, EPW)], dst_v)
    plsc.subcore_barrier()

    def start(c, rows_x, dval_x, sem_r, sem_d):
        pltpu.async_copy(y_hbm.at[src_v.at[pl.ds(c * K, K)]], rows_x, sem_r)
        pltpu.async_copy(dinv_hbm.at[dst_v.at[pl.ds(c * K, K)]], dval_x, sem_d)

    def finish(c, rows_x, dval_x, sem_r, sem_d):
        pltpu.make_async_copy(y_hbm.at[pl.ds(0, K)], rows_x, sem_r).wait()
        pltpu.sync_copy(rows_x, acc_sh.at[dst_v.at[pl.ds(c * K, K)]], add=True)
        pltpu.make_async_copy(dinv_hbm.at[pl.ds(0, K)], dval_x, sem_d).wait()
        pltpu.sync_copy(dval_x, t_sh.at[src_v.at[pl.ds(c * K, K)]], add=True)

    start(0, rowsA, dvalA, semA, semA2)

    def body(g, carry):
        start(2 * g + 1, rowsB, dvalB, semB, semB2)
        finish(2 * g, rowsA, dvalA, semA, semA2)
        start(2 * g + 2, rowsA, dvalA, semA, semA2)
        finish(2 * g + 1, rowsB, dvalB, semB, semB2)
        return carry

    lax.fori_loop(0, (NCHUNK - 1) // 2, body, 0)
    finish(NCHUNK - 1, rowsA, dvalA, semA, semA2)
    plsc.subcore_barrier()
    pltpu.sync_copy(acc_sh.at[pl.ds(base, RPT)],
                    acc_out.at[pl.ds(cid * NPAD + base, RPT)])
    pltpu.sync_copy(t_sh.at[pl.ds(base, RPT)],
                    t_out.at[pl.ds(cid * NPAD + base, RPT)])


def _scale_body(d0_ref, d1_ref, x_ref, w_ref, y_ref, dv_ref):
    deg = d0_ref[...] + d1_ref[...] + 1.0
    dinv = lax.rsqrt(deg)
    dv_ref[...] = dinv
    xw = jnp.dot(x_ref[...], w_ref[...], preferred_element_type=jnp.float32)
    y_ref[...] = xw * dinv


_scale = pl.pallas_call(
    _scale_body,
    grid=(GRID,),
    in_specs=[
        pl.BlockSpec((BM, 1), lambda i: (i, 0)),
        pl.BlockSpec((BM, 1), lambda i: (i, 0)),
        pl.BlockSpec((BM, D), lambda i: (i, 0)),
        pl.BlockSpec((D, D), lambda i: (0, 0)),
    ],
    out_specs=[
        pl.BlockSpec((BM, D), lambda i: (i, 0)),
        pl.BlockSpec((BM, 1), lambda i: (i, 0)),
    ],
    out_shape=[
        jax.ShapeDtypeStruct((NPAD, D), _f32),
        jax.ShapeDtypeStruct((NPAD, 1), _f32),
    ],
)


def _final_body(acc_ref, y_ref, dv_ref, t0_ref, t1_ref, b1_ref,
                W2_ref, b2_ref, Wr1_ref, br1_ref, Wr2_ref, br2_ref,
                Wu_ref, bu_ref, pred_ref, unc_ref, v_ref):
    i = pl.program_id(0)
    acc = acc_ref[0] + acc_ref[1] + y_ref[...]
    dinv = dv_ref[...]
    h = jnp.maximum(acc * dinv + b1_ref[...], 0.0)
    t = t0_ref[...] + t1_ref[...] + dinv
    c = dinv * t
    rowid = lax.broadcasted_iota(jnp.int32, (BM, 1), 0) + i * BM
    c = jnp.where(rowid < N, c, 0.0)
    part = jnp.sum(c * h, axis=0, keepdims=True)

    @pl.when(i == 0)
    def _():
        v_ref[...] = jnp.zeros_like(v_ref)

    v_ref[...] += part

    @pl.when(i == GRID - 1)
    def _():
        v = v_ref[...] * (1.0 / N)
        g = jnp.dot(v, W2_ref[...], preferred_element_type=jnp.float32)
        g = g + b2_ref[...]
        hid = jnp.dot(g, Wr1_ref[...], preferred_element_type=jnp.float32)
        hid = jnp.maximum(hid + br1_ref[...], 0.0)
        pred = jnp.dot(hid, Wr2_ref[...], preferred_element_type=jnp.float32)
        pred_ref[...] = pred + br2_ref[...]
        u = jnp.dot(g, Wu_ref[...], preferred_element_type=jnp.float32)
        u = u + bu_ref[...]
        unc_ref[...] = 1.0 / (1.0 + jnp.exp(-u))


def _full(shape):
    return pl.BlockSpec(shape, lambda i: tuple(0 for _ in shape))


_final = pl.pallas_call(
    _final_body,
    grid=(GRID,),
    in_specs=[
        pl.BlockSpec((NC, BM, D), lambda i: (0, i, 0)),
        pl.BlockSpec((BM, D), lambda i: (i, 0)),
        pl.BlockSpec((BM, 1), lambda i: (i, 0)),
        pl.BlockSpec((BM, 1), lambda i: (i, 0)),
        pl.BlockSpec((BM, 1), lambda i: (i, 0)),
        _full((1, D)),
        _full((D, D)),
        _full((1, D)),
        _full((D, D // 2)),
        _full((1, D // 2)),
        _full((D // 2, 1)),
        _full((1, 1)),
        _full((D, 1)),
        _full((1, 1)),
    ],
    out_specs=[_full((1, 1)), _full((1, 1))],
    out_shape=[jax.ShapeDtypeStruct((1, 1), _f32),
               jax.ShapeDtypeStruct((1, 1), _f32)],
    scratch_shapes=[pltpu.VMEM((1, D), _f32)],
)


def kernel(x, edge_index, W1, b1, W2, b2, Wr1, br1, Wr2, br2, Wu, bu):
    src2 = edge_index[0]
    dst2 = edge_index[1]
    x_pad = jnp.pad(x, ((0, NPAD - N), (0, 0)))

    deg2 = _deg_kernel(dst2)
    deg0 = deg2[:NPAD].reshape(NPAD, 1)
    deg1 = deg2[NPAD:].reshape(NPAD, 1)
    y, dinv_c = _scale(deg0, deg1, x_pad, W1)
    dinv = dinv_c.reshape(NPAD)

    zeros2d = jnp.zeros((NPAD, D), _f32)
    acc2, t2 = _scatter_kernel(src2, dst2, y, dinv, zeros2d)
    acc3 = acc2.reshape(NC, NPAD, D)
    t0 = t2[:NPAD].reshape(NPAD, 1)
    t1 = t2[NPAD:].reshape(NPAD, 1)

    pred, unc = _final(acc3, y, dinv_c, t0, t1,
                       b1.reshape(1, D), W2, b2.reshape(1, D),
                       Wr1, br1.reshape(1, D // 2), Wr2, br2.reshape(1, 1),
                       Wu, bu.reshape(1, 1))
    return (pred, unc)
